# baseline (speedup 1.0000x reference)
import functools

import jax
import jax.numpy as jnp
from jax.experimental import pallas as pl
from jax.experimental.pallas import tpu as pltpu


def _yw_body(x_ref, rm_ref, w1_ref, w2_ref, y1_ref, y2t_ref):
    rm = rm_ref[...]
    x = jnp.where(rm > 0.5, x_ref[...], 0.0).astype(jnp.bfloat16)
    w1 = w1_ref[...].astype(jnp.bfloat16)
    w2 = w2_ref[...].astype(jnp.bfloat16)
    y1_ref[...] = jax.lax.dot_general(
        x, w1, (((1,), (0,)), ((), ())),
        preferred_element_type=jnp.float32).astype(jnp.bfloat16)
    y2t_ref[...] = jax.lax.dot_general(
        w2, x, (((0,), (1,)), ((), ())),
        preferred_element_type=jnp.float32).astype(jnp.bfloat16)


def _main_body(ti, tj, a_ref, cm_ref, y1_ref, y2t_ref,
               out_in_ref, out_outt_ref, deg_r_ref):
    j = pl.program_id(0)
    i = pl.program_id(1)
    cm = cm_ref[...].reshape(1, tj)
    a = a_ref[...]
    ab = a.astype(jnp.bfloat16)
    y1 = y1_ref[...]
    isl = pl.ds(i * ti, ti)
    y2t = y2t_ref[...]

    c_in = jax.lax.dot_general(
        ab, y1, (((1,), (0,)), ((), ())), preferred_element_type=jnp.float32)
    c_outt = jax.lax.dot_general(
        y2t, ab, (((1,), (0,)), ((), ())), preferred_element_type=jnp.float32)
    rs = jnp.sum(a * cm, axis=1, keepdims=True)

    @pl.when(j == 0)
    def _():
        out_in_ref[isl, :] = c_in
        deg_r_ref[isl, :] = rs

    @pl.when(j > 0)
    def _():
        out_in_ref[isl, :] += c_in
        deg_r_ref[isl, :] += rs

    @pl.when(i == 0)
    def _():
        out_outt_ref[...] = c_outt

    @pl.when(i > 0)
    def _():
        out_outt_ref[...] += c_outt


def _epi_body(d_out, oi_ref, oot_ref, dr_ref, out_ref):
    r1 = 1.0 / jnp.clip(dr_ref[...], 1e-6, None)
    oota = oot_ref[...]
    r2 = 1.0 / jnp.clip(oota[d_out:, :], 1e-6, None)
    oot = oota[:d_out, :] * r2
    out_ref[...] = oi_ref[...] * r1 + oot.T


def kernel(X, A, W_in, W_out):
    n, d_in = X.shape
    d_out = W_in.shape[1]

    if n >= 2048:
        ti, tj = 1024, 2048
    else:
        ti = tj = n
    nj = -(-n // tj)
    ni = -(-n // ti)
    n_pad = nj * tj
    assert ni * ti == n_pad
    valid = jnp.pad(jnp.ones((n,), jnp.float32), (0, n_pad - n))
    col_valid = valid.reshape(nj, 1, tj)
    row_valid = valid.reshape(n_pad, 1)

    y1, y2t = pl.pallas_call(
        _yw_body,
        grid=(ni,),
        in_specs=[
            pl.BlockSpec((ti, d_in), lambda b: (b, 0)),
            pl.BlockSpec((ti, 1), lambda b: (b, 0)),
            pl.BlockSpec((d_in, d_out), lambda b: (0, 0)),
            pl.BlockSpec((d_in, d_out), lambda b: (0, 0)),
        ],
        out_specs=[
            pl.BlockSpec((ti, d_out), lambda b: (b, 0)),
            pl.BlockSpec((d_out, ti), lambda b: (0, b)),
        ],
        out_shape=[
            jax.ShapeDtypeStruct((n_pad, d_out), jnp.bfloat16),
            jax.ShapeDtypeStruct((d_out, n_pad), jnp.bfloat16),
        ],
    )(X, row_valid, W_in, W_out)

    y2t_aug = jnp.concatenate(
        [y2t, valid.reshape(1, n_pad).astype(jnp.bfloat16)], axis=0)
    out_in, out_outt, deg_r = pl.pallas_call(
        functools.partial(_main_body, ti, tj),
        grid=(nj, ni),
        in_specs=[
            pl.BlockSpec((ti, tj), lambda j, i: (i, j)),
            pl.BlockSpec((1, 1, tj), lambda j, i: (j, 0, 0)),
            pl.BlockSpec((tj, d_out), lambda j, i: (j, 0)),
            pl.BlockSpec((d_out + 1, ti), lambda j, i: (0, i)),
        ],
        out_specs=[
            pl.BlockSpec((n_pad, d_out), lambda j, i: (0, 0)),
            pl.BlockSpec((d_out + 1, tj), lambda j, i: (0, j)),
            pl.BlockSpec((n_pad, 1), lambda j, i: (0, 0)),
        ],
        out_shape=[
            jax.ShapeDtypeStruct((n_pad, d_out), jnp.float32),
            jax.ShapeDtypeStruct((d_out + 1, n_pad), jnp.float32),
            jax.ShapeDtypeStruct((n_pad, 1), jnp.float32),
        ],
        compiler_params=pltpu.CompilerParams(
            dimension_semantics=("arbitrary", "arbitrary"),
            vmem_limit_bytes=64 * 1024 * 1024,
        ),
    )(A, col_valid, y1, y2t_aug)

    te = 1920 if n >= 2048 else n
    out = pl.pallas_call(
        functools.partial(_epi_body, d_out),
        grid=(-(-n // te),),
        in_specs=[
            pl.BlockSpec((te, d_out), lambda b: (b, 0)),
            pl.BlockSpec((d_out + 1, te), lambda b: (0, b)),
            pl.BlockSpec((te, 1), lambda b: (b, 0)),
        ],
        out_specs=pl.BlockSpec((te, d_out), lambda b: (b, 0)),
        out_shape=jax.ShapeDtypeStruct((n, d_out), jnp.float32),
    )(out_in, out_outt, deg_r)
    return out
